# Initial kernel scaffold; baseline (speedup 1.0000x reference)
import jax
import jax.numpy as jnp
from jax.experimental import pallas as pl


def kernel(epoch, CircRNAs, Drugs, edge_index, circRNA_index, drug_index, edge_weight, drugdata, params):
    raise NotImplementedError("write your pallas kernel here")


# speedup vs baseline: 15.5834x; 15.5834x over previous
import functools

import jax
import jax.numpy as jnp
from jax import lax
from jax.experimental import pallas as pl
from jax.experimental.pallas import tpu as pltpu

N_CIRC = 271
N_DRUG = 218
N_NODES = 489
N_EDGES = 20000
N_PAIRS = 59078

E_PAD = 20480
P_PAD = 59392
C_PAD = 272
D_PAD = 224
CB = 8
N_CBLK = C_PAD // CB
ROWS_BLK = CB * D_PAD


def _mm(a, b):
    return jax.lax.dot_general(a, b, (((1,), (0,)), ((), ())),
                               preferred_element_type=jnp.float32)


def _mm_tn(a, b):
    return jax.lax.dot_general(a, b, (((0,), (0,)), ((), ())),
                               preferred_element_type=jnp.float32)


def _onehot_chunk(ref, i, n_classes):
    start = pl.multiple_of(i * 16, 16)
    blk = ref[pl.ds(start, 16), :]
    iota = lax.broadcasted_iota(jnp.int32, (16, 128, n_classes), 2)
    oh = (blk[:, :, None] == iota).astype(jnp.float32)
    return oh.reshape(2048, n_classes)


def _graph_kernel(nodes_ref, srce_ref, dste_ref, cip_ref, dikp_ref,
                  bn0g_ref, bn0b_ref, ln0g_ref, ln0b_ref,
                  W1_ref, as1_ref, ad1T_ref, b1_ref,
                  W2_ref, as2_ref, ad2T_ref, b2_ref,
                  bn1g_ref, bn1b_ref, ln1g_ref, ln1b_ref,
                  Wd0a_ref, Wd0b_ref,
                  A_ref, Bf_ref, w_ref):
    f32 = jnp.float32
    nodes = nodes_ref[...]
    m = jnp.mean(nodes, axis=0, keepdims=True)
    xc = nodes - m
    v = jnp.mean(xc * xc, axis=0, keepdims=True)
    nodes = xc / jnp.sqrt(v + 1e-5) * bn0g_ref[...] + bn0b_ref[...]
    mr = jnp.mean(nodes, axis=1, keepdims=True)
    xr = nodes - mr
    vr = jnp.mean(xr * xr, axis=1, keepdims=True)
    nodes = xr / jnp.sqrt(vr + 1e-5) * ln0g_ref[...] + ln0b_ref[...]

    ii = lax.broadcasted_iota(jnp.int32, (N_NODES, N_NODES), 0)
    jj = lax.broadcasted_iota(jnp.int32, (N_NODES, N_NODES), 1)
    cnt0 = (ii == jj).astype(f32)

    def cnt_body(i, acc):
        ohs = _onehot_chunk(srce_ref, i, N_NODES)
        ohd = _onehot_chunk(dste_ref, i, N_NODES)
        return acc + _mm_tn(ohd, ohs)

    Cnt = lax.fori_loop(0, E_PAD // 2048, cnt_body, cnt0)
    mask = Cnt > 0.0

    def w_body(i, acc):
        ohc = _onehot_chunk(cip_ref, i, C_PAD)
        ohd = _onehot_chunk(dikp_ref, i, D_PAD)
        return acc + _mm_tn(ohc, ohd)

    w_ref[...] = lax.fori_loop(0, P_PAD // 2048, w_body,
                               jnp.zeros((C_PAD, D_PAD), f32))

    def gat(x, W_ref_, as_ref_, adT_ref_, b_ref_, heads, out_ch):
        h = _mm(x, W_ref_[...])
        outs = []
        for hh in range(heads):
            hv = h[:, hh * out_ch:(hh + 1) * out_ch]
            hvT = hv.T
            as_row = _mm(as_ref_[hh:hh + 1, :], hvT)
            ad_col = _mm(hv, adT_ref_[:, hh:hh + 1])
            E = ad_col + as_row
            E = jnp.where(E >= 0.0, E, 0.2 * E)
            emax = jnp.max(jnp.where(mask, E, -jnp.inf), axis=1, keepdims=True)
            ex = jnp.where(mask, jnp.exp(E - emax), 0.0)
            exC = Cnt * ex
            den = jnp.sum(exC, axis=1, keepdims=True)
            Wt = exC / (den + 1e-16)
            outs.append(_mm(Wt, hv))
        o = outs[0] if heads == 1 else jnp.concatenate(outs, axis=1)
        return o + b_ref_[...]

    x1 = jax.nn.relu(gat(nodes, W1_ref, as1_ref, ad1T_ref, b1_ref, 8, 64))
    x2 = jax.nn.relu(gat(x1, W2_ref, as2_ref, ad2T_ref, b2_ref, 1, 256))

    m2 = jnp.mean(x2, axis=0, keepdims=True)
    xc2 = x2 - m2
    v2 = jnp.mean(xc2 * xc2, axis=0, keepdims=True)
    x2 = xc2 / jnp.sqrt(v2 + 1e-5) * bn1g_ref[...] + bn1b_ref[...]
    mr2 = jnp.mean(x2, axis=1, keepdims=True)
    xr2 = x2 - mr2
    vr2 = jnp.mean(xr2 * xr2, axis=1, keepdims=True)
    x = xr2 / jnp.sqrt(vr2 + 1e-5) * ln1g_ref[...] + ln1b_ref[...]

    A_ref[...] = _mm(x, Wd0a_ref[...])
    Bf_ref[...] = _mm(x, Wd0b_ref[...])


def _stats1_kernel(A_ref, B_ref, w_ref, bd0_ref, sum_ref, sq_ref):
    @pl.when(pl.program_id(0) == 0)
    def _():
        sum_ref[...] = jnp.zeros_like(sum_ref)
        sq_ref[...] = jnp.zeros_like(sq_ref)

    z = A_ref[...][:, None, :] + B_ref[...][None, :, :] + bd0_ref[...][None, :, :]
    r = jnp.maximum(z, 0.0)
    wf = w_ref[...][:, :, None]
    wr = wf * r
    sum_ref[...] += jnp.sum(jnp.sum(wr, axis=0), axis=0, keepdims=True)
    sq_ref[...] += jnp.sum(jnp.sum(wr * r, axis=0), axis=0, keepdims=True)


def _main_kernel(A_ref, B_ref, w_ref, bd0_ref, sum1_ref, sq1_ref,
                 g1_ref, bb1_ref, Wd1_ref, bd1_ref,
                 h2r_ref, sum2_ref, sq2_ref):
    @pl.when(pl.program_id(0) == 0)
    def _():
        sum2_ref[...] = jnp.zeros_like(sum2_ref)
        sq2_ref[...] = jnp.zeros_like(sq2_ref)

    m1 = sum1_ref[...] / N_PAIRS
    v1 = sq1_ref[...] / N_PAIRS - m1 * m1
    s1 = g1_ref[...] / jnp.sqrt(v1 + 1e-5)
    t1 = bb1_ref[...] - m1 * s1

    z = A_ref[...][:, None, :] + B_ref[...][None, :, :] + bd0_ref[...][None, :, :]
    y = jnp.maximum(z, 0.0) * s1[None, :, :] + t1[None, :, :]
    h2 = _mm(y.reshape(ROWS_BLK, 512), Wd1_ref[...]) + bd1_ref[...]
    r2 = jnp.maximum(h2, 0.0)
    h2r_ref[...] = r2
    r3 = r2.reshape(CB, D_PAD, 128)
    wf = w_ref[...][:, :, None]
    wr = wf * r3
    sum2_ref[...] += jnp.sum(jnp.sum(wr, axis=0), axis=0, keepdims=True)
    sq2_ref[...] += jnp.sum(jnp.sum(wr * r3, axis=0), axis=0, keepdims=True)


def _final_kernel(h2r_ref, sum2_ref, sq2_ref, g2_ref, bb2_ref,
                  woutT_ref, bout_ref, R_ref):
    m2 = sum2_ref[...] / N_PAIRS
    v2 = sq2_ref[...] / N_PAIRS - m2 * m2
    s2 = g2_ref[...] / jnp.sqrt(v2 + 1e-5)
    t2 = bb2_ref[...] - m2 * s2
    wv = s2 * woutT_ref[...]
    const = jnp.sum(t2 * woutT_ref[...], axis=1, keepdims=True) + bout_ref[...]
    r3 = h2r_ref[...].reshape(CB, D_PAD, 128)
    logit = jnp.sum(r3 * wv[None, :, :], axis=-1) + const
    R_ref[...] = 1.0 / (1.0 + jnp.exp(-logit))


def _gather_kernel(cip_ref, dikp_ref, R_ref, out_ref):
    cif = cip_ref[...]
    dif = dikp_ref[...]
    iota_c = lax.broadcasted_iota(jnp.int32, (16, 128, C_PAD), 2)
    ohc = (cif[:, :, None] == iota_c).astype(jnp.float32).reshape(2048, C_PAD)
    g1 = _mm(ohc, R_ref[...])
    iota_d = lax.broadcasted_iota(jnp.int32, (16, 128, D_PAD), 2)
    ohd = (dif[:, :, None] == iota_d).astype(jnp.float32)
    out_ref[...] = jnp.sum(g1.reshape(16, 128, D_PAD) * ohd, axis=-1)


def kernel(epoch, CircRNAs, Drugs, edge_index, circRNA_index, drug_index,
           edge_weight, drugdata, params):
    p = params
    f32 = jnp.float32
    i32 = jnp.int32

    nodes_raw = jnp.concatenate([CircRNAs[:, :N_NODES], Drugs[:, :N_NODES]], axis=0)
    srce = jnp.pad(edge_index[0].astype(i32), (0, E_PAD - N_EDGES),
                   constant_values=-1).reshape(E_PAD // 128, 128)
    dste = jnp.pad(edge_index[1].astype(i32), (0, E_PAD - N_EDGES),
                   constant_values=-1).reshape(E_PAD // 128, 128)
    cip = jnp.pad(circRNA_index.astype(i32), (0, P_PAD - N_PAIRS),
                  constant_values=-1).reshape(P_PAD // 128, 128)
    dikp = jnp.pad(drug_index.astype(i32) - N_CIRC, (0, P_PAD - N_PAIRS),
                   constant_values=-1).reshape(P_PAD // 128, 128)

    row = lambda a: a.reshape(1, -1).astype(f32)

    A, Bf, w = pl.pallas_call(
        _graph_kernel,
        out_shape=[
            jax.ShapeDtypeStruct((N_NODES, 512), f32),
            jax.ShapeDtypeStruct((N_NODES, 512), f32),
            jax.ShapeDtypeStruct((C_PAD, D_PAD), f32),
        ],
    )(nodes_raw, srce, dste, cip, dikp,
      row(p["bn0_g"]), row(p["bn0_b"]), row(p["ln0_g"]), row(p["ln0_b"]),
      p["W1"], p["a_src1"], p["a_dst1"].T, row(p["b1"]),
      p["W2"], p["a_src2"], p["a_dst2"].T, row(p["b2"]),
      row(p["bn1_g"]), row(p["bn1_b"]), row(p["ln1_g"]), row(p["ln1_b"]),
      p["Wd0"][:256], p["Wd0"][256:])

    Ac = jnp.pad(A[:N_CIRC], ((0, C_PAD - N_CIRC), (0, 0)))
    Bd = jnp.pad(Bf[N_CIRC:], ((0, D_PAD - N_DRUG), (0, 0)))

    blkA = pl.BlockSpec((CB, 512), lambda i: (i, 0))
    blkB = pl.BlockSpec((D_PAD, 512), lambda i: (0, 0))
    blkw = pl.BlockSpec((CB, D_PAD), lambda i: (i, 0))
    acc512 = pl.BlockSpec((1, 512), lambda i: (0, 0))
    acc128 = pl.BlockSpec((1, 128), lambda i: (0, 0))
    row512 = pl.BlockSpec((1, 512), lambda i: (0, 0))
    row128 = pl.BlockSpec((1, 128), lambda i: (0, 0))

    sum1, sq1 = pl.pallas_call(
        _stats1_kernel,
        grid=(N_CBLK,),
        in_specs=[blkA, blkB, blkw, row512],
        out_specs=[acc512, acc512],
        out_shape=[jax.ShapeDtypeStruct((1, 512), f32)] * 2,
    )(Ac, Bd, w, row(p["bd0"]))

    h2r, sum2, sq2 = pl.pallas_call(
        _main_kernel,
        grid=(N_CBLK,),
        in_specs=[blkA, blkB, blkw, row512, row512, row512, row512, row512,
                  pl.BlockSpec((512, 128), lambda i: (0, 0)), row128],
        out_specs=[pl.BlockSpec((ROWS_BLK, 128), lambda i: (i, 0)), acc128, acc128],
        out_shape=[
            jax.ShapeDtypeStruct((C_PAD * D_PAD, 128), f32),
            jax.ShapeDtypeStruct((1, 128), f32),
            jax.ShapeDtypeStruct((1, 128), f32),
        ],
    )(Ac, Bd, w, row(p["bd0"]), sum1, sq1,
      row(p["bnd0_g"]), row(p["bnd0_b"]), p["Wd1"], row(p["bd1"]))

    R = pl.pallas_call(
        _final_kernel,
        grid=(N_CBLK,),
        in_specs=[pl.BlockSpec((ROWS_BLK, 128), lambda i: (i, 0)),
                  row128, row128, row128, row128, row128,
                  pl.BlockSpec((1, 1), lambda i: (0, 0))],
        out_specs=pl.BlockSpec((CB, D_PAD), lambda i: (i, 0)),
        out_shape=jax.ShapeDtypeStruct((C_PAD, D_PAD), f32),
    )(h2r, sum2, sq2, row(p["bnd1_g"]), row(p["bnd1_b"]),
      p["Wout"].T, p["bout"].reshape(1, 1))

    out2d = pl.pallas_call(
        _gather_kernel,
        grid=(P_PAD // 2048,),
        in_specs=[pl.BlockSpec((16, 128), lambda i: (i, 0)),
                  pl.BlockSpec((16, 128), lambda i: (i, 0)),
                  pl.BlockSpec((C_PAD, D_PAD), lambda i: (0, 0))],
        out_specs=pl.BlockSpec((16, 128), lambda i: (i, 0)),
        out_shape=jax.ShapeDtypeStruct((P_PAD // 128, 128), f32),
    )(cip, dikp, R)

    return out2d.reshape(-1)[:N_PAIRS]


# speedup vs baseline: 16.0505x; 1.0300x over previous
import functools

import jax
import jax.numpy as jnp
from jax import lax
from jax.experimental import pallas as pl
from jax.experimental.pallas import tpu as pltpu
from jax.experimental.pallas import tpu_sc as plsc

N_CIRC = 271
N_DRUG = 218
N_NODES = 489
N_EDGES = 20000
N_PAIRS = 59078

E_PAD = 20480
P_PAD = 59392
C_PAD = 272
D_PAD = 224
CB = 8
N_CBLK = C_PAD // CB
ROWS_BLK = CB * D_PAD


def _mm(a, b):
    return jax.lax.dot_general(a, b, (((1,), (0,)), ((), ())),
                               preferred_element_type=jnp.float32)


def _mm_tn(a, b):
    return jax.lax.dot_general(a, b, (((0,), (0,)), ((), ())),
                               preferred_element_type=jnp.float32)


def _onehot_chunk(ref, i, n_classes):
    start = pl.multiple_of(i * 16, 16)
    blk = ref[pl.ds(start, 16), :]
    iota = lax.broadcasted_iota(jnp.int32, (16, 128, n_classes), 2)
    oh = (blk[:, :, None] == iota).astype(jnp.bfloat16)
    return oh.reshape(2048, n_classes)


def _graph_kernel(nodes_ref, srce_ref, dste_ref, cip_ref, dikp_ref,
                  bn0g_ref, bn0b_ref, ln0g_ref, ln0b_ref,
                  W1_ref, as1_ref, ad1T_ref, b1_ref,
                  W2_ref, as2_ref, ad2T_ref, b2_ref,
                  bn1g_ref, bn1b_ref, ln1g_ref, ln1b_ref,
                  Wd0a_ref, Wd0b_ref,
                  A_ref, Bf_ref, w_ref):
    f32 = jnp.float32
    nodes = nodes_ref[...]
    m = jnp.mean(nodes, axis=0, keepdims=True)
    xc = nodes - m
    v = jnp.mean(xc * xc, axis=0, keepdims=True)
    nodes = xc / jnp.sqrt(v + 1e-5) * bn0g_ref[...] + bn0b_ref[...]
    mr = jnp.mean(nodes, axis=1, keepdims=True)
    xr = nodes - mr
    vr = jnp.mean(xr * xr, axis=1, keepdims=True)
    nodes = xr / jnp.sqrt(vr + 1e-5) * ln0g_ref[...] + ln0b_ref[...]

    ii = lax.broadcasted_iota(jnp.int32, (N_NODES, N_NODES), 0)
    jj = lax.broadcasted_iota(jnp.int32, (N_NODES, N_NODES), 1)
    cnt0 = (ii == jj).astype(f32)

    def cnt_body(i, acc):
        ohs = _onehot_chunk(srce_ref, i, N_NODES)
        ohd = _onehot_chunk(dste_ref, i, N_NODES)
        return acc + _mm_tn(ohd, ohs)

    Cnt = lax.fori_loop(0, E_PAD // 2048, cnt_body, cnt0)
    mask = Cnt > 0.0

    def w_body(i, acc):
        ohc = _onehot_chunk(cip_ref, i, C_PAD)
        ohd = _onehot_chunk(dikp_ref, i, D_PAD)
        return acc + _mm_tn(ohc, ohd)

    w_ref[...] = lax.fori_loop(0, P_PAD // 2048, w_body,
                               jnp.zeros((C_PAD, D_PAD), f32))

    def gat(x, W_ref_, as_ref_, adT_ref_, b_ref_, heads, out_ch):
        h = _mm(x, W_ref_[...])
        outs = []
        for hh in range(heads):
            hv = h[:, hh * out_ch:(hh + 1) * out_ch]
            hvT = hv.T
            as_row = _mm(as_ref_[hh:hh + 1, :], hvT)
            ad_col = _mm(hv, adT_ref_[:, hh:hh + 1])
            E = ad_col + as_row
            E = jnp.where(E >= 0.0, E, 0.2 * E)
            emax = jnp.max(jnp.where(mask, E, -jnp.inf), axis=1, keepdims=True)
            ex = jnp.where(mask, jnp.exp(E - emax), 0.0)
            exC = Cnt * ex
            den = jnp.sum(exC, axis=1, keepdims=True)
            Wt = exC / (den + 1e-16)
            outs.append(_mm(Wt, hv))
        o = outs[0] if heads == 1 else jnp.concatenate(outs, axis=1)
        return o + b_ref_[...]

    x1 = jax.nn.relu(gat(nodes, W1_ref, as1_ref, ad1T_ref, b1_ref, 8, 64))
    x2 = jax.nn.relu(gat(x1, W2_ref, as2_ref, ad2T_ref, b2_ref, 1, 256))

    m2 = jnp.mean(x2, axis=0, keepdims=True)
    xc2 = x2 - m2
    v2 = jnp.mean(xc2 * xc2, axis=0, keepdims=True)
    x2 = xc2 / jnp.sqrt(v2 + 1e-5) * bn1g_ref[...] + bn1b_ref[...]
    mr2 = jnp.mean(x2, axis=1, keepdims=True)
    xr2 = x2 - mr2
    vr2 = jnp.mean(xr2 * xr2, axis=1, keepdims=True)
    x = xr2 / jnp.sqrt(vr2 + 1e-5) * ln1g_ref[...] + ln1b_ref[...]

    A_ref[...] = _mm(x, Wd0a_ref[...])
    Bf_ref[...] = _mm(x, Wd0b_ref[...])


def _stats1_kernel(A_ref, B_ref, w_ref, bd0_ref, sum_ref, sq_ref):
    @pl.when(pl.program_id(0) == 0)
    def _():
        sum_ref[...] = jnp.zeros_like(sum_ref)
        sq_ref[...] = jnp.zeros_like(sq_ref)

    z = A_ref[...][:, None, :] + B_ref[...][None, :, :] + bd0_ref[...][None, :, :]
    r = jnp.maximum(z, 0.0)
    wf = w_ref[...][:, :, None]
    wr = wf * r
    sum_ref[...] += jnp.sum(jnp.sum(wr, axis=0), axis=0, keepdims=True)
    sq_ref[...] += jnp.sum(jnp.sum(wr * r, axis=0), axis=0, keepdims=True)


def _main_kernel(A_ref, B_ref, w_ref, bd0_ref, sum1_ref, sq1_ref,
                 g1_ref, bb1_ref, Wd1_ref, bd1_ref,
                 h2r_ref, sum2_ref, sq2_ref):
    @pl.when(pl.program_id(0) == 0)
    def _():
        sum2_ref[...] = jnp.zeros_like(sum2_ref)
        sq2_ref[...] = jnp.zeros_like(sq2_ref)

    m1 = sum1_ref[...] / N_PAIRS
    v1 = sq1_ref[...] / N_PAIRS - m1 * m1
    s1 = g1_ref[...] / jnp.sqrt(v1 + 1e-5)
    t1 = bb1_ref[...] - m1 * s1

    z = A_ref[...][:, None, :] + B_ref[...][None, :, :] + bd0_ref[...][None, :, :]
    y = jnp.maximum(z, 0.0) * s1[None, :, :] + t1[None, :, :]
    h2 = _mm(y.reshape(ROWS_BLK, 512), Wd1_ref[...]) + bd1_ref[...]
    r2 = jnp.maximum(h2, 0.0)
    h2r_ref[...] = r2
    r3 = r2.reshape(CB, D_PAD, 128)
    wf = w_ref[...][:, :, None]
    wr = wf * r3
    sum2_ref[...] += jnp.sum(jnp.sum(wr, axis=0), axis=0, keepdims=True)
    sq2_ref[...] += jnp.sum(jnp.sum(wr * r3, axis=0), axis=0, keepdims=True)


def _final_kernel(h2r_ref, sum2_ref, sq2_ref, g2_ref, bb2_ref,
                  woutT_ref, bout_ref, R_ref):
    m2 = sum2_ref[...] / N_PAIRS
    v2 = sq2_ref[...] / N_PAIRS - m2 * m2
    s2 = g2_ref[...] / jnp.sqrt(v2 + 1e-5)
    t2 = bb2_ref[...] - m2 * s2
    wv = s2 * woutT_ref[...]
    const = jnp.sum(t2 * woutT_ref[...], axis=1, keepdims=True) + bout_ref[...]
    r3 = h2r_ref[...].reshape(CB, D_PAD, 128)
    logit = jnp.sum(r3 * wv[None, :, :], axis=-1) + const
    R_ref[...] = 1.0 / (1.0 + jnp.exp(-logit))


SC_NC = 2
SC_NS = 16
SC_NW = SC_NC * SC_NS
SC_BPW = P_PAD // SC_NW


def _sc_gather_body(ci_hbm, dik_hbm, R_hbm, out_hbm, idx_v, dik_v, rows_v, sem):
    wid = lax.axis_index("s") * SC_NC + lax.axis_index("c")
    base = wid * SC_BPW
    pltpu.sync_copy(ci_hbm.at[pl.ds(base, SC_BPW)], idx_v)
    pltpu.sync_copy(dik_hbm.at[pl.ds(base, SC_BPW)], dik_v)

    def body(i, carry):
        s = pl.ds(i * 16, 16)
        idx_v[s] = idx_v[s] * D_PAD + dik_v[s]
        return carry

    lax.fori_loop(0, SC_BPW // 16, body, 0)
    pltpu.async_copy(R_hbm.at[idx_v], rows_v, sem).wait()
    pltpu.sync_copy(rows_v, out_hbm.at[pl.ds(base, SC_BPW)])


def kernel(epoch, CircRNAs, Drugs, edge_index, circRNA_index, drug_index,
           edge_weight, drugdata, params):
    p = params
    f32 = jnp.float32
    i32 = jnp.int32

    nodes_raw = jnp.concatenate([CircRNAs[:, :N_NODES], Drugs[:, :N_NODES]], axis=0)
    srce = jnp.pad(edge_index[0].astype(i32), (0, E_PAD - N_EDGES),
                   constant_values=-1).reshape(E_PAD // 128, 128)
    dste = jnp.pad(edge_index[1].astype(i32), (0, E_PAD - N_EDGES),
                   constant_values=-1).reshape(E_PAD // 128, 128)
    cip = jnp.pad(circRNA_index.astype(i32), (0, P_PAD - N_PAIRS),
                  constant_values=-1).reshape(P_PAD // 128, 128)
    dikp = jnp.pad(drug_index.astype(i32) - N_CIRC, (0, P_PAD - N_PAIRS),
                   constant_values=-1).reshape(P_PAD // 128, 128)

    row = lambda a: a.reshape(1, -1).astype(f32)

    A, Bf, w = pl.pallas_call(
        _graph_kernel,
        out_shape=[
            jax.ShapeDtypeStruct((N_NODES, 512), f32),
            jax.ShapeDtypeStruct((N_NODES, 512), f32),
            jax.ShapeDtypeStruct((C_PAD, D_PAD), f32),
        ],
    )(nodes_raw, srce, dste, cip, dikp,
      row(p["bn0_g"]), row(p["bn0_b"]), row(p["ln0_g"]), row(p["ln0_b"]),
      p["W1"], p["a_src1"], p["a_dst1"].T, row(p["b1"]),
      p["W2"], p["a_src2"], p["a_dst2"].T, row(p["b2"]),
      row(p["bn1_g"]), row(p["bn1_b"]), row(p["ln1_g"]), row(p["ln1_b"]),
      p["Wd0"][:256], p["Wd0"][256:])

    Ac = jnp.pad(A[:N_CIRC], ((0, C_PAD - N_CIRC), (0, 0)))
    Bd = jnp.pad(Bf[N_CIRC:], ((0, D_PAD - N_DRUG), (0, 0)))

    blkA = pl.BlockSpec((CB, 512), lambda i: (i, 0))
    blkB = pl.BlockSpec((D_PAD, 512), lambda i: (0, 0))
    blkw = pl.BlockSpec((CB, D_PAD), lambda i: (i, 0))
    acc512 = pl.BlockSpec((1, 512), lambda i: (0, 0))
    acc128 = pl.BlockSpec((1, 128), lambda i: (0, 0))
    row512 = pl.BlockSpec((1, 512), lambda i: (0, 0))
    row128 = pl.BlockSpec((1, 128), lambda i: (0, 0))

    sum1, sq1 = pl.pallas_call(
        _stats1_kernel,
        grid=(N_CBLK,),
        in_specs=[blkA, blkB, blkw, row512],
        out_specs=[acc512, acc512],
        out_shape=[jax.ShapeDtypeStruct((1, 512), f32)] * 2,
    )(Ac, Bd, w, row(p["bd0"]))

    h2r, sum2, sq2 = pl.pallas_call(
        _main_kernel,
        grid=(N_CBLK,),
        in_specs=[blkA, blkB, blkw, row512, row512, row512, row512, row512,
                  pl.BlockSpec((512, 128), lambda i: (0, 0)), row128],
        out_specs=[pl.BlockSpec((ROWS_BLK, 128), lambda i: (i, 0)), acc128, acc128],
        out_shape=[
            jax.ShapeDtypeStruct((C_PAD * D_PAD, 128), f32),
            jax.ShapeDtypeStruct((1, 128), f32),
            jax.ShapeDtypeStruct((1, 128), f32),
        ],
    )(Ac, Bd, w, row(p["bd0"]), sum1, sq1,
      row(p["bnd0_g"]), row(p["bnd0_b"]), p["Wd1"], row(p["bd1"]))

    R = pl.pallas_call(
        _final_kernel,
        grid=(N_CBLK,),
        in_specs=[pl.BlockSpec((ROWS_BLK, 128), lambda i: (i, 0)),
                  row128, row128, row128, row128, row128,
                  pl.BlockSpec((1, 1), lambda i: (0, 0))],
        out_specs=pl.BlockSpec((CB, D_PAD), lambda i: (i, 0)),
        out_shape=jax.ShapeDtypeStruct((C_PAD, D_PAD), f32),
    )(h2r, sum2, sq2, row(p["bnd1_g"]), row(p["bnd1_b"]),
      p["Wout"].T, p["bout"].reshape(1, 1))

    ci_sc = jnp.pad(circRNA_index.astype(i32), (0, P_PAD - N_PAIRS))
    dik_sc = jnp.pad(drug_index.astype(i32), (0, P_PAD - N_PAIRS),
                     constant_values=N_CIRC) - N_CIRC
    out1d = pl.kernel(
        _sc_gather_body,
        out_type=jax.ShapeDtypeStruct((P_PAD,), f32),
        mesh=plsc.VectorSubcoreMesh(core_axis_name="c", subcore_axis_name="s",
                                    num_cores=SC_NC, num_subcores=SC_NS),
        scratch_types=[
            pltpu.VMEM((SC_BPW,), i32),
            pltpu.VMEM((SC_BPW,), i32),
            pltpu.VMEM((SC_BPW,), f32),
            pltpu.SemaphoreType.DMA,
        ],
    )(ci_sc, dik_sc, R.reshape(-1))

    return out1d[:N_PAIRS]


# speedup vs baseline: 21.3803x; 1.3321x over previous
import functools

import jax
import jax.numpy as jnp
from jax import lax
from jax.experimental import pallas as pl
from jax.experimental.pallas import tpu as pltpu
from jax.experimental.pallas import tpu_sc as plsc

N_CIRC = 271
N_DRUG = 218
N_NODES = 489
N_EDGES = 20000
N_PAIRS = 59078

E_PAD = 20480
P_PAD = 59392
C_PAD = 272
D_PAD = 224
CB = 8
N_CBLK = C_PAD // CB
ROWS_BLK = CB * D_PAD


def _mm(a, b):
    return jax.lax.dot_general(a, b, (((1,), (0,)), ((), ())),
                               preferred_element_type=jnp.float32)


def _graph_kernel(nodes_ref, cnt_ref,
                  bn0g_ref, bn0b_ref, ln0g_ref, ln0b_ref,
                  W1_ref, as1_ref, ad1T_ref, b1_ref,
                  W2_ref, as2_ref, ad2T_ref, b2_ref,
                  bn1g_ref, bn1b_ref, ln1g_ref, ln1b_ref,
                  Wd0a_ref, Wd0b_ref,
                  A_ref, Bf_ref):
    f32 = jnp.float32
    nodes = nodes_ref[...]
    m = jnp.mean(nodes, axis=0, keepdims=True)
    xc = nodes - m
    v = jnp.mean(xc * xc, axis=0, keepdims=True)
    nodes = xc / jnp.sqrt(v + 1e-5) * bn0g_ref[...] + bn0b_ref[...]
    mr = jnp.mean(nodes, axis=1, keepdims=True)
    xr = nodes - mr
    vr = jnp.mean(xr * xr, axis=1, keepdims=True)
    nodes = xr / jnp.sqrt(vr + 1e-5) * ln0g_ref[...] + ln0b_ref[...]

    ii = lax.broadcasted_iota(jnp.int32, (N_NODES, N_NODES), 0)
    jj = lax.broadcasted_iota(jnp.int32, (N_NODES, N_NODES), 1)
    Cnt = cnt_ref[...][:, :N_NODES] + (ii == jj).astype(f32)
    mask = Cnt > 0.0

    def gat(x, W_ref_, as_ref_, adT_ref_, b_ref_, heads, out_ch):
        h = _mm(x, W_ref_[...])
        outs = []
        for hh in range(heads):
            hv = h[:, hh * out_ch:(hh + 1) * out_ch]
            hvT = hv.T
            as_row = _mm(as_ref_[hh:hh + 1, :], hvT)
            ad_col = _mm(hv, adT_ref_[:, hh:hh + 1])
            E = ad_col + as_row
            E = jnp.where(E >= 0.0, E, 0.2 * E)
            emax = jnp.max(jnp.where(mask, E, -jnp.inf), axis=1, keepdims=True)
            ex = jnp.where(mask, jnp.exp(E - emax), 0.0)
            exC = Cnt * ex
            den = jnp.sum(exC, axis=1, keepdims=True)
            Wt = exC / (den + 1e-16)
            outs.append(_mm(Wt, hv))
        o = outs[0] if heads == 1 else jnp.concatenate(outs, axis=1)
        return o + b_ref_[...]

    x1 = jax.nn.relu(gat(nodes, W1_ref, as1_ref, ad1T_ref, b1_ref, 8, 64))
    x2 = jax.nn.relu(gat(x1, W2_ref, as2_ref, ad2T_ref, b2_ref, 1, 256))

    m2 = jnp.mean(x2, axis=0, keepdims=True)
    xc2 = x2 - m2
    v2 = jnp.mean(xc2 * xc2, axis=0, keepdims=True)
    x2 = xc2 / jnp.sqrt(v2 + 1e-5) * bn1g_ref[...] + bn1b_ref[...]
    mr2 = jnp.mean(x2, axis=1, keepdims=True)
    xr2 = x2 - mr2
    vr2 = jnp.mean(xr2 * xr2, axis=1, keepdims=True)
    x = xr2 / jnp.sqrt(vr2 + 1e-5) * ln1g_ref[...] + ln1b_ref[...]

    zc = jnp.zeros((C_PAD - N_CIRC, 256), f32)
    zd = jnp.zeros((D_PAD - N_DRUG, 256), f32)
    xc_pad = jnp.concatenate([x[:N_CIRC], zc], axis=0)
    xd_pad = jnp.concatenate([x[N_CIRC:], zd], axis=0)
    A_ref[...] = _mm(xc_pad, Wd0a_ref[...])
    Bf_ref[...] = _mm(xd_pad, Wd0b_ref[...])


def _stats1_kernel(A_ref, B_ref, wrow_ref, bd0_ref, sum_ref, sq_ref):
    @pl.when(pl.program_id(0) == 0)
    def _():
        sum_ref[...] = jnp.zeros_like(sum_ref)
        sq_ref[...] = jnp.zeros_like(sq_ref)

    z = A_ref[...][:, None, :] + B_ref[...][None, :, :] + bd0_ref[...][None, :, :]
    r = jnp.maximum(z, 0.0).reshape(ROWS_BLK, 512)
    wrow = wrow_ref[...]
    sum_ref[...] += _mm(wrow, r)
    sq_ref[...] += _mm(wrow, r * r)


def _main_kernel(A_ref, B_ref, wrow_ref, bd0_ref, sum1_ref, sq1_ref,
                 g1_ref, bb1_ref, Wd1_ref, bd1_ref,
                 h2r_ref, sum2_ref, sq2_ref):
    @pl.when(pl.program_id(0) == 0)
    def _():
        sum2_ref[...] = jnp.zeros_like(sum2_ref)
        sq2_ref[...] = jnp.zeros_like(sq2_ref)

    m1 = sum1_ref[...] / N_PAIRS
    v1 = sq1_ref[...] / N_PAIRS - m1 * m1
    s1 = g1_ref[...] / jnp.sqrt(v1 + 1e-5)
    t1 = bb1_ref[...] - m1 * s1
    c1 = _mm(t1.astype(jnp.bfloat16), Wd1_ref[...]) + bd1_ref[...]

    z = A_ref[...][:, None, :] + B_ref[...][None, :, :] + bd0_ref[...][None, :, :]
    y = (jnp.maximum(z, 0.0) * s1[None, :, :]).astype(jnp.bfloat16)
    h2 = _mm(y.reshape(ROWS_BLK, 512), Wd1_ref[...]) + c1
    r2 = jnp.maximum(h2, 0.0)
    h2r_ref[...] = r2.astype(jnp.bfloat16)
    wrow = wrow_ref[...]
    sum2_ref[...] += _mm(wrow, r2)
    sq2_ref[...] += _mm(wrow, r2 * r2)


def _final_kernel(h2r_ref, sum2_ref, sq2_ref, g2_ref, bb2_ref,
                  woutT_ref, bout_ref, R_ref):
    m2 = sum2_ref[...] / N_PAIRS
    v2 = sq2_ref[...] / N_PAIRS - m2 * m2
    s2 = g2_ref[...] / jnp.sqrt(v2 + 1e-5)
    t2 = bb2_ref[...] - m2 * s2
    wv = s2 * woutT_ref[...]
    const = jnp.sum(t2 * woutT_ref[...], axis=1, keepdims=True) + bout_ref[...]
    r3 = h2r_ref[...].astype(jnp.float32).reshape(CB, D_PAD, 128)
    logit = jnp.sum(r3 * wv[None, :, :], axis=-1) + const
    R_ref[...] = 1.0 / (1.0 + jnp.exp(-logit))


SC_NC = 2
SC_NS = 16
SC_NW = SC_NC * SC_NS
SC_BPW = P_PAD // SC_NW

CNT_N = N_NODES * 512
CNT_CH = CNT_N // SC_NS
W_N = C_PAD * D_PAD + 256
W_CH = W_N // SC_NS
E_ROWS = E_PAD // 128 // SC_NS
P_ROWS = P_PAD // 128 // SC_NS


def _sc_hist_body(src_hbm, dst_hbm, ciw_hbm, dikw_hbm,
                  cnt_hbm, w_hbm, es_v, ed_v, pc_v, pd_v, e2_v, p2_v, ones_v,
                  zb_v, cnt_sh, w_sh):
    cid = lax.axis_index("c")
    sid = lax.axis_index("s")

    @pl.when(cid == 0)
    def _():
        ne = E_ROWS * 128
        np_ = P_ROWS * 128
        pltpu.sync_copy(src_hbm.at[pl.ds(sid * ne, ne)], es_v)
        pltpu.sync_copy(dst_hbm.at[pl.ds(sid * ne, ne)], ed_v)
        pltpu.sync_copy(ciw_hbm.at[pl.ds(sid * np_, np_)], pc_v)
        pltpu.sync_copy(dikw_hbm.at[pl.ds(sid * np_, np_)], pd_v)
        for k in range(8):
            ones_v[pl.ds(k * 16, 16)] = jnp.ones((16,), jnp.float32)

        def zbody(i, c):
            zb_v[pl.ds(i * 16, 16)] = jnp.zeros((16,), jnp.float32)
            return c

        lax.fori_loop(0, (CNT_CH // 2) // 16, zbody, 0)
        pltpu.sync_copy(zb_v, cnt_sh.at[pl.ds(sid * CNT_CH, CNT_CH // 2)])
        pltpu.sync_copy(zb_v, cnt_sh.at[pl.ds(sid * CNT_CH + CNT_CH // 2,
                                              CNT_CH // 2)])
        pltpu.sync_copy(zb_v.at[pl.ds(0, W_CH)],
                        w_sh.at[pl.ds(sid * W_CH, W_CH)])
        for j in range(E_ROWS):
            for k in range(8):
                s = pl.ds(j * 128 + k * 16, 16)
                e2_v[j, pl.ds(k * 16, 16)] = ed_v[s] * 512 + es_v[s]
        for j in range(P_ROWS):
            for k in range(8):
                s = pl.ds(j * 128 + k * 16, 16)
                p2_v[j, pl.ds(k * 16, 16)] = pc_v[s] * D_PAD + pd_v[s]
        plsc.subcore_barrier()
        for j in range(E_ROWS):
            pltpu.sync_copy(ones_v, cnt_sh.at[e2_v.at[j]], add=True)
        for j in range(P_ROWS):
            pltpu.sync_copy(ones_v, w_sh.at[p2_v.at[j]], add=True)
        plsc.subcore_barrier()
        half = CNT_CH // 2
        for part in range(2):
            off = sid * CNT_CH + part * half
            pltpu.sync_copy(cnt_sh.at[pl.ds(off, half)], zb_v)
            pltpu.sync_copy(zb_v, cnt_hbm.at[pl.ds(off, half)])
        pltpu.sync_copy(w_sh.at[pl.ds(sid * W_CH, W_CH)],
                        zb_v.at[pl.ds(0, W_CH)])
        pltpu.sync_copy(zb_v.at[pl.ds(0, W_CH)],
                        w_hbm.at[pl.ds(sid * W_CH, W_CH)])


def _sc_gather_body(ci_hbm, dik_hbm, R_hbm, out_hbm, idx_v, dik_v, rows_v, sem):
    wid = lax.axis_index("s") * SC_NC + lax.axis_index("c")
    base = wid * SC_BPW
    pltpu.sync_copy(ci_hbm.at[pl.ds(base, SC_BPW)], idx_v)
    pltpu.sync_copy(dik_hbm.at[pl.ds(base, SC_BPW)], dik_v)

    def body(i, carry):
        s = pl.ds(i * 16, 16)
        idx_v[s] = idx_v[s] * D_PAD + dik_v[s]
        return carry

    lax.fori_loop(0, SC_BPW // 16, body, 0)
    pltpu.async_copy(R_hbm.at[idx_v], rows_v, sem).wait()
    pltpu.sync_copy(rows_v, out_hbm.at[pl.ds(base, SC_BPW)])


def kernel(epoch, CircRNAs, Drugs, edge_index, circRNA_index, drug_index,
           edge_weight, drugdata, params):
    p = params
    f32 = jnp.float32
    i32 = jnp.int32

    nodes_raw = jnp.concatenate([CircRNAs[:, :N_NODES], Drugs[:, :N_NODES]], axis=0)
    srce = jnp.pad(edge_index[0].astype(i32), (0, E_PAD - N_EDGES),
                   constant_values=500)
    dste = jnp.pad(edge_index[1].astype(i32), (0, E_PAD - N_EDGES))
    ciw = jnp.pad(circRNA_index.astype(i32), (0, P_PAD - N_PAIRS),
                  constant_values=C_PAD)
    dikw = jnp.pad(drug_index.astype(i32), (0, P_PAD - N_PAIRS),
                   constant_values=N_CIRC) - N_CIRC

    cnt_flat, w_flat = pl.kernel(
        _sc_hist_body,
        out_type=[jax.ShapeDtypeStruct((CNT_N,), f32),
                  jax.ShapeDtypeStruct((W_N,), f32)],
        mesh=plsc.VectorSubcoreMesh(core_axis_name="c", subcore_axis_name="s",
                                    num_cores=SC_NC, num_subcores=SC_NS),
        scratch_types=[
            pltpu.VMEM((E_ROWS * 128,), i32),
            pltpu.VMEM((E_ROWS * 128,), i32),
            pltpu.VMEM((P_ROWS * 128,), i32),
            pltpu.VMEM((P_ROWS * 128,), i32),
            pltpu.VMEM((E_ROWS, 128), i32),
            pltpu.VMEM((P_ROWS, 128), i32),
            pltpu.VMEM((128,), f32),
            pltpu.VMEM((CNT_CH // 2,), f32),
            pltpu.VMEM_SHARED((CNT_N,), f32),
            pltpu.VMEM_SHARED((W_N,), f32),
        ],
    )(srce, dste, ciw, dikw)

    cnt2d = cnt_flat.reshape(N_NODES, 512)
    wrow = w_flat[:C_PAD * D_PAD].reshape(1, C_PAD * D_PAD)

    row = lambda a: a.reshape(1, -1).astype(f32)

    Ac, Bd = pl.pallas_call(
        _graph_kernel,
        out_shape=[
            jax.ShapeDtypeStruct((C_PAD, 512), f32),
            jax.ShapeDtypeStruct((D_PAD, 512), f32),
        ],
    )(nodes_raw, cnt2d,
      row(p["bn0_g"]), row(p["bn0_b"]), row(p["ln0_g"]), row(p["ln0_b"]),
      p["W1"], p["a_src1"], p["a_dst1"].T, row(p["b1"]),
      p["W2"], p["a_src2"], p["a_dst2"].T, row(p["b2"]),
      row(p["bn1_g"]), row(p["bn1_b"]), row(p["ln1_g"]), row(p["ln1_b"]),
      p["Wd0"][:256], p["Wd0"][256:])

    blkA = pl.BlockSpec((CB, 512), lambda i: (i, 0))
    blkB = pl.BlockSpec((D_PAD, 512), lambda i: (0, 0))
    blkw = pl.BlockSpec((1, ROWS_BLK), lambda i: (0, i))
    acc512 = pl.BlockSpec((1, 512), lambda i: (0, 0))
    acc128 = pl.BlockSpec((1, 128), lambda i: (0, 0))
    row512 = pl.BlockSpec((1, 512), lambda i: (0, 0))
    row128 = pl.BlockSpec((1, 128), lambda i: (0, 0))

    sum1, sq1 = pl.pallas_call(
        _stats1_kernel,
        grid=(N_CBLK,),
        in_specs=[blkA, blkB, blkw, row512],
        out_specs=[acc512, acc512],
        out_shape=[jax.ShapeDtypeStruct((1, 512), f32)] * 2,
    )(Ac, Bd, wrow, row(p["bd0"]))

    h2r, sum2, sq2 = pl.pallas_call(
        _main_kernel,
        grid=(N_CBLK,),
        in_specs=[blkA, blkB, blkw, row512, row512, row512, row512, row512,
                  pl.BlockSpec((512, 128), lambda i: (0, 0)), row128],
        out_specs=[pl.BlockSpec((ROWS_BLK, 128), lambda i: (i, 0)), acc128, acc128],
        out_shape=[
            jax.ShapeDtypeStruct((C_PAD * D_PAD, 128), jnp.bfloat16),
            jax.ShapeDtypeStruct((1, 128), f32),
            jax.ShapeDtypeStruct((1, 128), f32),
        ],
    )(Ac, Bd, wrow, row(p["bd0"]), sum1, sq1,
      row(p["bnd0_g"]), row(p["bnd0_b"]), p["Wd1"].astype(jnp.bfloat16),
      row(p["bd1"]))

    R = pl.pallas_call(
        _final_kernel,
        grid=(N_CBLK,),
        in_specs=[pl.BlockSpec((ROWS_BLK, 128), lambda i: (i, 0)),
                  row128, row128, row128, row128, row128,
                  pl.BlockSpec((1, 1), lambda i: (0, 0))],
        out_specs=pl.BlockSpec((CB, D_PAD), lambda i: (i, 0)),
        out_shape=jax.ShapeDtypeStruct((C_PAD, D_PAD), f32),
    )(h2r, sum2, sq2, row(p["bnd1_g"]), row(p["bnd1_b"]),
      p["Wout"].T, p["bout"].reshape(1, 1))

    ci_sc = jnp.pad(circRNA_index.astype(i32), (0, P_PAD - N_PAIRS))
    dik_sc = jnp.pad(drug_index.astype(i32), (0, P_PAD - N_PAIRS),
                     constant_values=N_CIRC) - N_CIRC
    out1d = pl.kernel(
        _sc_gather_body,
        out_type=jax.ShapeDtypeStruct((P_PAD,), f32),
        mesh=plsc.VectorSubcoreMesh(core_axis_name="c", subcore_axis_name="s",
                                    num_cores=SC_NC, num_subcores=SC_NS),
        scratch_types=[
            pltpu.VMEM((SC_BPW,), i32),
            pltpu.VMEM((SC_BPW,), i32),
            pltpu.VMEM((SC_BPW,), f32),
            pltpu.SemaphoreType.DMA,
        ],
    )(ci_sc, dik_sc, R.reshape(-1))

    return out1d[:N_PAIRS]


# speedup vs baseline: 25.2535x; 1.1812x over previous
import functools

import jax
import jax.numpy as jnp
from jax import lax
from jax.experimental import pallas as pl
from jax.experimental.pallas import tpu as pltpu
from jax.experimental.pallas import tpu_sc as plsc

N_CIRC = 271
N_DRUG = 218
N_NODES = 489
N_EDGES = 20000
N_PAIRS = 59078

E_PAD = 20480
P_PAD = 59392
C_PAD = 272
D_PAD = 224
CB = 8
N_CBLK = C_PAD // CB
ROWS_BLK = CB * D_PAD


def _mm(a, b):
    return jax.lax.dot_general(a, b, (((1,), (0,)), ((), ())),
                               preferred_element_type=jnp.float32)


def _mega_kernel(nodes_ref, cnt_ref, wrow_ref,
                 bn0g_ref, bn0b_ref, ln0g_ref, ln0b_ref,
                 W1_ref, as1_ref, ad1T_ref, b1_ref,
                 W2_ref, as2_ref, ad2T_ref, b2_ref,
                 bn1g_ref, bn1b_ref, ln1g_ref, ln1b_ref,
                 Wd0a_ref, Wd0b_ref, bd0_ref,
                 Wd1_ref, bd1_ref, g1_ref, bb1_ref,
                 g2_ref, bb2_ref, woutT_ref, bout_ref,
                 R_ref, a_scr, b_scr, r2_scr):
    f32 = jnp.float32
    nodes = nodes_ref[...]
    m = jnp.mean(nodes, axis=0, keepdims=True)
    xc = nodes - m
    v = jnp.mean(xc * xc, axis=0, keepdims=True)
    nodes = xc / jnp.sqrt(v + 1e-5) * bn0g_ref[...] + bn0b_ref[...]
    mr = jnp.mean(nodes, axis=1, keepdims=True)
    xr = nodes - mr
    vr = jnp.mean(xr * xr, axis=1, keepdims=True)
    nodes = xr / jnp.sqrt(vr + 1e-5) * ln0g_ref[...] + ln0b_ref[...]

    ii = lax.broadcasted_iota(jnp.int32, (N_NODES, N_NODES), 0)
    jj = lax.broadcasted_iota(jnp.int32, (N_NODES, N_NODES), 1)
    Cnt = cnt_ref[...][:, :N_NODES] + (ii == jj).astype(f32)
    mask = Cnt > 0.0

    def gat(x, W_ref_, as_ref_, adT_ref_, b_ref_, heads, out_ch):
        h = _mm(x, W_ref_[...])
        outs = []
        for hh in range(heads):
            hv = h[:, hh * out_ch:(hh + 1) * out_ch]
            hvT = hv.T
            as_row = _mm(as_ref_[hh:hh + 1, :], hvT)
            ad_col = _mm(hv, adT_ref_[:, hh:hh + 1])
            E = ad_col + as_row
            E = jnp.where(E >= 0.0, E, 0.2 * E)
            emax = jnp.max(jnp.where(mask, E, -jnp.inf), axis=1, keepdims=True)
            ex = jnp.where(mask, jnp.exp(E - emax), 0.0)
            exC = Cnt * ex
            den = jnp.sum(exC, axis=1, keepdims=True)
            Wt = exC / (den + 1e-16)
            outs.append(_mm(Wt, hv))
        o = outs[0] if heads == 1 else jnp.concatenate(outs, axis=1)
        return o + b_ref_[...]

    x1 = jax.nn.relu(gat(nodes, W1_ref, as1_ref, ad1T_ref, b1_ref, 8, 64))
    x2 = jax.nn.relu(gat(x1, W2_ref, as2_ref, ad2T_ref, b2_ref, 1, 256))

    m2 = jnp.mean(x2, axis=0, keepdims=True)
    xc2 = x2 - m2
    v2 = jnp.mean(xc2 * xc2, axis=0, keepdims=True)
    x2 = xc2 / jnp.sqrt(v2 + 1e-5) * bn1g_ref[...] + bn1b_ref[...]
    mr2 = jnp.mean(x2, axis=1, keepdims=True)
    xr2 = x2 - mr2
    vr2 = jnp.mean(xr2 * xr2, axis=1, keepdims=True)
    x = xr2 / jnp.sqrt(vr2 + 1e-5) * ln1g_ref[...] + ln1b_ref[...]

    zc = jnp.zeros((C_PAD - N_CIRC, 256), f32)
    zd = jnp.zeros((D_PAD - N_DRUG, 256), f32)
    xc_pad = jnp.concatenate([x[:N_CIRC], zc], axis=0)
    xd_pad = jnp.concatenate([x[N_CIRC:], zd], axis=0)
    a_scr[...] = _mm(xc_pad, Wd0a_ref[...]) + bd0_ref[...]
    b_scr[...] = _mm(xd_pad, Wd0b_ref[...])

    def blk(i):
        c0 = pl.multiple_of(i * CB, CB)
        r0 = pl.multiple_of(i * ROWS_BLK, ROWS_BLK)
        z = a_scr[pl.ds(c0, CB), :][:, None, :] + b_scr[...][None, :, :]
        wr = wrow_ref[0:1, pl.ds(r0, ROWS_BLK)]
        return z, wr, r0

    def s1_body(i, carry):
        su, sq = carry
        z, wr, _ = blk(i)
        r = jnp.maximum(z, 0.0).reshape(ROWS_BLK, 512)
        return su + _mm(wr, r), sq + _mm(wr, r * r)

    zr = jnp.zeros((1, 512), f32)
    sum1, sq1 = lax.fori_loop(0, N_CBLK, s1_body, (zr, zr))
    m1 = sum1 / N_PAIRS
    v1 = sq1 / N_PAIRS - m1 * m1
    s1 = g1_ref[...] / jnp.sqrt(v1 + 1e-5)
    t1 = bb1_ref[...] - m1 * s1
    st = jnp.concatenate([s1, t1, s1, s1, s1, s1, s1, s1], axis=0)
    s1_col = st.T[:, 0:1]
    W1s = (s1_col * Wd1_ref[...]).astype(jnp.bfloat16)
    c1 = _mm(t1, Wd1_ref[...]) + bd1_ref[...]

    def s2_body(i, carry):
        su, sq = carry
        z, wr, r0 = blk(i)
        yb = jnp.maximum(z, 0.0).astype(jnp.bfloat16).reshape(ROWS_BLK, 512)
        h2 = _mm(yb, W1s) + c1
        r2 = jnp.maximum(h2, 0.0)
        r2_scr[pl.ds(r0, ROWS_BLK), :] = r2.astype(jnp.bfloat16)
        return su + _mm(wr, r2), sq + _mm(wr, r2 * r2)

    zr2 = jnp.zeros((1, 128), f32)
    sum2, sq2 = lax.fori_loop(0, N_CBLK, s2_body, (zr2, zr2))
    m2 = sum2 / N_PAIRS
    v2 = sq2 / N_PAIRS - m2 * m2
    s2 = g2_ref[...] / jnp.sqrt(v2 + 1e-5)
    t2 = bb2_ref[...] - m2 * s2
    wv = s2 * woutT_ref[...]
    const = jnp.sum(t2 * woutT_ref[...], axis=1, keepdims=True) + bout_ref[...]

    def out_body(i, carry):
        c0 = pl.multiple_of(i * CB, CB)
        r0 = pl.multiple_of(i * ROWS_BLK, ROWS_BLK)
        r3 = r2_scr[pl.ds(r0, ROWS_BLK), :].astype(f32).reshape(CB, D_PAD, 128)
        logit = jnp.sum(r3 * wv[None, :, :], axis=-1) + const
        R_ref[pl.ds(c0, CB), :] = 1.0 / (1.0 + jnp.exp(-logit))
        return carry

    lax.fori_loop(0, N_CBLK, out_body, 0)


SC_NC = 2
SC_NS = 16
SC_NW = SC_NC * SC_NS
SC_BPW = P_PAD // SC_NW

CNT_N = N_NODES * 512
CNT_CH = CNT_N // SC_NS
W_N = C_PAD * D_PAD + 256
W_CH = W_N // SC_NS
E_ROWS = E_PAD // 128 // SC_NS
P_ROWS = P_PAD // 128 // SC_NS


def _sc_hist_body(src_hbm, dst_hbm, ciw_hbm, dikw_hbm,
                  cnt_hbm, w_hbm, es_v, ed_v, pc_v, pd_v, e2_v, p2_v, ones_v,
                  zb_v, cnt_sh, w_sh):
    cid = lax.axis_index("c")
    sid = lax.axis_index("s")

    @pl.when(cid == 0)
    def _():
        ne = E_ROWS * 128
        np_ = P_ROWS * 128
        pltpu.sync_copy(src_hbm.at[pl.ds(sid * ne, ne)], es_v)
        pltpu.sync_copy(dst_hbm.at[pl.ds(sid * ne, ne)], ed_v)
        pltpu.sync_copy(ciw_hbm.at[pl.ds(sid * np_, np_)], pc_v)
        pltpu.sync_copy(dikw_hbm.at[pl.ds(sid * np_, np_)], pd_v)
        for k in range(8):
            ones_v[pl.ds(k * 16, 16)] = jnp.ones((16,), jnp.float32)

        def zbody(i, c):
            zb_v[pl.ds(i * 16, 16)] = jnp.zeros((16,), jnp.float32)
            return c

        lax.fori_loop(0, (CNT_CH // 2) // 16, zbody, 0)
        pltpu.sync_copy(zb_v, cnt_sh.at[pl.ds(sid * CNT_CH, CNT_CH // 2)])
        pltpu.sync_copy(zb_v, cnt_sh.at[pl.ds(sid * CNT_CH + CNT_CH // 2,
                                              CNT_CH // 2)])
        pltpu.sync_copy(zb_v.at[pl.ds(0, W_CH)],
                        w_sh.at[pl.ds(sid * W_CH, W_CH)])
        for j in range(E_ROWS):
            for k in range(8):
                s = pl.ds(j * 128 + k * 16, 16)
                e2_v[j, pl.ds(k * 16, 16)] = ed_v[s] * 512 + es_v[s]
        for j in range(P_ROWS):
            for k in range(8):
                s = pl.ds(j * 128 + k * 16, 16)
                p2_v[j, pl.ds(k * 16, 16)] = pc_v[s] * D_PAD + pd_v[s]
        plsc.subcore_barrier()
        for j in range(E_ROWS):
            pltpu.sync_copy(ones_v, cnt_sh.at[e2_v.at[j]], add=True)
        for j in range(P_ROWS):
            pltpu.sync_copy(ones_v, w_sh.at[p2_v.at[j]], add=True)
        plsc.subcore_barrier()
        half = CNT_CH // 2
        for part in range(2):
            off = sid * CNT_CH + part * half
            pltpu.sync_copy(cnt_sh.at[pl.ds(off, half)], zb_v)
            pltpu.sync_copy(zb_v, cnt_hbm.at[pl.ds(off, half)])
        pltpu.sync_copy(w_sh.at[pl.ds(sid * W_CH, W_CH)],
                        zb_v.at[pl.ds(0, W_CH)])
        pltpu.sync_copy(zb_v.at[pl.ds(0, W_CH)],
                        w_hbm.at[pl.ds(sid * W_CH, W_CH)])


def _sc_gather_body(ci_hbm, dik_hbm, R_hbm, out_hbm, idx_v, dik_v, rows_v, sem):
    wid = lax.axis_index("s") * SC_NC + lax.axis_index("c")
    base = wid * SC_BPW
    pltpu.sync_copy(ci_hbm.at[pl.ds(base, SC_BPW)], idx_v)
    pltpu.sync_copy(dik_hbm.at[pl.ds(base, SC_BPW)], dik_v)

    def body(i, carry):
        s = pl.ds(i * 16, 16)
        idx_v[s] = idx_v[s] * D_PAD + dik_v[s]
        return carry

    lax.fori_loop(0, SC_BPW // 16, body, 0)
    pltpu.async_copy(R_hbm.at[idx_v], rows_v, sem).wait()
    pltpu.sync_copy(rows_v, out_hbm.at[pl.ds(base, SC_BPW)])


def kernel(epoch, CircRNAs, Drugs, edge_index, circRNA_index, drug_index,
           edge_weight, drugdata, params):
    p = params
    f32 = jnp.float32
    i32 = jnp.int32

    nodes_raw = jnp.concatenate([CircRNAs[:, :N_NODES], Drugs[:, :N_NODES]], axis=0)
    srce = jnp.pad(edge_index[0].astype(i32), (0, E_PAD - N_EDGES),
                   constant_values=500)
    dste = jnp.pad(edge_index[1].astype(i32), (0, E_PAD - N_EDGES))
    ciw = jnp.pad(circRNA_index.astype(i32), (0, P_PAD - N_PAIRS),
                  constant_values=C_PAD)
    dikw = jnp.pad(drug_index.astype(i32), (0, P_PAD - N_PAIRS),
                   constant_values=N_CIRC) - N_CIRC

    cnt_flat, w_flat = pl.kernel(
        _sc_hist_body,
        out_type=[jax.ShapeDtypeStruct((CNT_N,), f32),
                  jax.ShapeDtypeStruct((W_N,), f32)],
        mesh=plsc.VectorSubcoreMesh(core_axis_name="c", subcore_axis_name="s",
                                    num_cores=SC_NC, num_subcores=SC_NS),
        scratch_types=[
            pltpu.VMEM((E_ROWS * 128,), i32),
            pltpu.VMEM((E_ROWS * 128,), i32),
            pltpu.VMEM((P_ROWS * 128,), i32),
            pltpu.VMEM((P_ROWS * 128,), i32),
            pltpu.VMEM((E_ROWS, 128), i32),
            pltpu.VMEM((P_ROWS, 128), i32),
            pltpu.VMEM((128,), f32),
            pltpu.VMEM((CNT_CH // 2,), f32),
            pltpu.VMEM_SHARED((CNT_N,), f32),
            pltpu.VMEM_SHARED((W_N,), f32),
        ],
    )(srce, dste, ciw, dikw)

    cnt2d = cnt_flat.reshape(N_NODES, 512)
    wrow = w_flat[:C_PAD * D_PAD].reshape(1, C_PAD * D_PAD)

    row = lambda a: a.reshape(1, -1).astype(f32)

    R = pl.pallas_call(
        _mega_kernel,
        out_shape=jax.ShapeDtypeStruct((C_PAD, D_PAD), f32),
        scratch_shapes=[
            pltpu.VMEM((C_PAD, 512), f32),
            pltpu.VMEM((D_PAD, 512), f32),
            pltpu.VMEM((C_PAD * D_PAD, 128), jnp.bfloat16),
        ],
    )(nodes_raw, cnt2d, wrow,
      row(p["bn0_g"]), row(p["bn0_b"]), row(p["ln0_g"]), row(p["ln0_b"]),
      p["W1"], p["a_src1"], p["a_dst1"].T, row(p["b1"]),
      p["W2"], p["a_src2"], p["a_dst2"].T, row(p["b2"]),
      row(p["bn1_g"]), row(p["bn1_b"]), row(p["ln1_g"]), row(p["ln1_b"]),
      p["Wd0"][:256], p["Wd0"][256:], row(p["bd0"]),
      p["Wd1"], row(p["bd1"]), row(p["bnd0_g"]), row(p["bnd0_b"]),
      row(p["bnd1_g"]), row(p["bnd1_b"]), p["Wout"].T, p["bout"].reshape(1, 1))

    ci_sc = jnp.pad(circRNA_index.astype(i32), (0, P_PAD - N_PAIRS))
    dik_sc = jnp.pad(drug_index.astype(i32), (0, P_PAD - N_PAIRS),
                     constant_values=N_CIRC) - N_CIRC
    out1d = pl.kernel(
        _sc_gather_body,
        out_type=jax.ShapeDtypeStruct((P_PAD,), f32),
        mesh=plsc.VectorSubcoreMesh(core_axis_name="c", subcore_axis_name="s",
                                    num_cores=SC_NC, num_subcores=SC_NS),
        scratch_types=[
            pltpu.VMEM((SC_BPW,), i32),
            pltpu.VMEM((SC_BPW,), i32),
            pltpu.VMEM((SC_BPW,), f32),
            pltpu.SemaphoreType.DMA,
        ],
    )(ci_sc, dik_sc, R.reshape(-1))

    return out1d[:N_PAIRS]
